# baseline (speedup 1.0000x reference)
import functools

import jax
import jax.numpy as jnp
from jax import lax
from jax.experimental import pallas as pl
from jax.experimental.pallas import tpu as pltpu
from jax.experimental.pallas import tpu_sc as plsc

D = 32
B = 16384
L = 16

_info = plsc.get_sparse_core_info()
NC = _info.num_cores
NS = _info.num_subcores
NW = NC * NS
BPW = B // NW
GROUPS = BPW // L

MARGIN = 1.0
EPS = 1e-6


def _sqrt16(x):
    x = jnp.maximum(x, jnp.float32(1e-30))
    i = lax.bitcast_convert_type(x, jnp.int32)
    i = jnp.int32(0x1FBD1DF5) + lax.shift_right_arithmetic(i, 1)
    y = lax.bitcast_convert_type(i, jnp.float32)
    for _ in range(3):
        y = jnp.float32(0.5) * (y + x / y)
    return y


HB = BPW // 2
HG = GROUPS // 2
WP = 33


def _tec_body(a_hbm, p_hbm, n_hbm, w_hbm, out_hbm,
              idx_a, idx_p, idx_n, g_ea, g_ep, g_en, p_ea, p_ep, p_en,
              part_v, sem):
    wid = lax.axis_index("s") * NC + lax.axis_index("c")
    base = wid * BPW

    lanes = lax.iota(jnp.int32, L)
    loss_vec = jnp.zeros((L,), jnp.float32)

    for h in range(2):
        bh = base + h * HB
        pltpu.sync_copy(a_hbm.at[pl.ds(bh, HB)], idx_a)
        pltpu.sync_copy(p_hbm.at[pl.ds(bh * 3, HB * 3)], idx_p)
        pltpu.sync_copy(n_hbm.at[pl.ds(bh * 3, HB * 3)], idx_n)
        cp_a = pltpu.async_copy(w_hbm.at[idx_a], g_ea, sem)
        cp_p = pltpu.async_copy(w_hbm.at[idx_p], g_ep, sem)
        cp_n = pltpu.async_copy(w_hbm.at[idx_n], g_en, sem)
        cp_a.wait()
        cp_p.wait()
        cp_n.wait()

        for src_ref, dst_ref, nrows in ((g_ea, p_ea, HB),
                                        (g_ep, p_ep, HB * 3),
                                        (g_en, p_en, HB * 3)):
            @plsc.parallel_loop(0, nrows, step=4, unroll=8)
            def _(r, _src=src_ref, _dst=dst_ref):
                for u in range(4):
                    rr = r + u
                    _dst[pl.ds(rr * WP, L)] = _src[rr, pl.ds(0, L)]
                    _dst[pl.ds(rr * WP + L, L)] = _src[rr, pl.ds(L, L)]

        @plsc.parallel_loop(0, HG, carry=loss_vec)
        def loss_vec(g, acc_loss):
            rows_a = (g * L + lanes) * WP
            rows3 = (g * L + lanes) * 3 * WP
            xrows = (rows3, rows3 + WP, rows3 + 2 * WP,
                     rows3, rows3 + WP, rows3 + 2 * WP)
            xrefs = (p_ep, p_ep, p_ep, p_en, p_en, p_en)
            acc = [jnp.zeros((L,), jnp.float32) for _ in range(6)]
            for d in range(D):
                ea_d = plsc.load_gather(p_ea, [rows_a + d]) + jnp.float32(EPS)
                for j in range(6):
                    t = ea_d - plsc.load_gather(xrefs[j], [xrows[j] + d])
                    acc[j] = acc[j] + t * t
            dist = [_sqrt16(acc[j]) for j in range(6)]
            for k in range(5):
                acc_loss = acc_loss + jnp.maximum(
                    dist[k] - dist[k + 1] + jnp.float32(MARGIN),
                    jnp.float32(0.0))
            return acc_loss

    part_v[...] = loss_vec
    pltpu.sync_copy(part_v, out_hbm.at[wid])


CPW = 244
CHUNK = 4
STEPS = CPW // CHUNK
CN = CHUNK * 128
JG = CN // L
SP = 33
TAIL_START = CPW * NW * 128
TAIL_ROWS = 1000000 - TAIL_START
TPR = TAIL_ROWS // NW


def _detile_body(w3_hbm, tail_hbm, out_hbm, in0_v, in1_v, pad_v, pk0_v, pk1_v,
                 tail_v, isem0, isem1, osem0, osem1):
    ins = (in0_v, in1_v)
    pks = (pk0_v, pk1_v)
    isems = (isem0, isem1)
    osems = (osem0, osem1)
    wid = lax.axis_index("s") * NC + lax.axis_index("c")
    col0 = wid * CPW
    lanes_sp = lax.iota(jnp.int32, L) * SP

    def fire_in(g, b):
        c = (col0 + g * CHUNK) * 128
        pltpu.async_copy(w3_hbm.at[:, :, pl.ds(c, CN)], ins[b], isems[b])

    def wait_in(g, b):
        c = (col0 + g * CHUNK) * 128
        pltpu.make_async_copy(w3_hbm.at[:, :, pl.ds(c, CN)], ins[b],
                              isems[b]).wait()

    def fire_out(g, b):
        c = (col0 + g * CHUNK) * 128
        pltpu.async_copy(pk_v := pks[b], out_hbm.at[pl.ds(c * D, CN * D)],
                         osems[b])

    def wait_out(g, b):
        c = (col0 + g * CHUNK) * 128
        pltpu.make_async_copy(pks[b], out_hbm.at[pl.ds(c * D, CN * D)],
                              osems[b]).wait()

    def compute(b):
        @plsc.parallel_loop(0, JG, unroll=16)
        def _(j):
            jb = lanes_sp + j * (L * SP)
            for s in range(4):
                for d in range(8):
                    v = ins[b][s, d, pl.ds(j * L, L)]
                    plsc.store_scatter(pad_v, [jb + (s * 8 + d)], v)

        @plsc.parallel_loop(0, CN, step=4, unroll=4)
        def _(n):
            for u in range(4):
                nn = n + u
                pks[b][pl.ds(nn * D, L)] = pad_v[pl.ds(nn * SP, L)]
                pks[b][pl.ds(nn * D + L, L)] = pad_v[pl.ds(nn * SP + L, L)]

    fire_in(0, 0)
    fire_in(1, 1)

    def step2(i, carry):
        g = i * 2
        for b in range(2):
            ge = g + b
            wait_in(ge, b)

            @pl.when(ge >= 2)
            def _():
                wait_out(ge - 2, b)

            compute(b)
            fire_out(ge, b)

            @pl.when(ge + 2 < STEPS)
            def _():
                fire_in(ge + 2, b)
        return carry

    lax.fori_loop(0, (STEPS - 1) // 2, step2, 0)
    wait_in(STEPS - 1, 0)
    wait_out(STEPS - 3, 0)
    compute(0)
    fire_out(STEPS - 1, 0)
    wait_out(STEPS - 2, 1)
    wait_out(STEPS - 1, 0)
    pltpu.sync_copy(tail_hbm.at[pl.ds(wid * TPR * D, TPR * D)], tail_v)
    pltpu.sync_copy(tail_v,
                    out_hbm.at[pl.ds((TAIL_START + wid * TPR) * D, TPR * D)])


@functools.partial(jax.jit, static_argnums=())
def _detile(w3, tail_flat):
    mesh = plsc.VectorSubcoreMesh(core_axis_name="c", subcore_axis_name="s")
    f = pl.kernel(
        _detile_body,
        mesh=mesh,
        compiler_params=pltpu.CompilerParams(needs_layout_passes=False),
        out_type=jax.ShapeDtypeStruct((1000000 * D,), jnp.float32),
        scratch_types=[
            pltpu.VMEM((4, 8, CN), jnp.float32),
            pltpu.VMEM((4, 8, CN), jnp.float32),
            pltpu.VMEM((CN * SP,), jnp.float32),
            pltpu.VMEM((CN * D,), jnp.float32),
            pltpu.VMEM((CN * D,), jnp.float32),
            pltpu.VMEM((TPR * D,), jnp.float32),
            pltpu.SemaphoreType.DMA,
            pltpu.SemaphoreType.DMA,
            pltpu.SemaphoreType.DMA,
            pltpu.SemaphoreType.DMA,
        ],
    )
    return f(w3, tail_flat)


@functools.partial(jax.jit, static_argnums=())
def _partial_losses(a, p_flat, n_flat, w):
    mesh = plsc.VectorSubcoreMesh(core_axis_name="c", subcore_axis_name="s")
    f = pl.kernel(
        _tec_body,
        mesh=mesh,
        compiler_params=pltpu.CompilerParams(
            needs_layout_passes=False, use_tc_tiling_on_sc=False),
        out_type=jax.ShapeDtypeStruct((NW, L), jnp.float32),
        scratch_types=[
            pltpu.VMEM((HB,), jnp.int32),
            pltpu.VMEM((HB * 3,), jnp.int32),
            pltpu.VMEM((HB * 3,), jnp.int32),
            pltpu.VMEM((HB, D), jnp.float32),
            pltpu.VMEM((HB * 3, D), jnp.float32),
            pltpu.VMEM((HB * 3, D), jnp.float32),
            pltpu.VMEM((HB * WP,), jnp.float32),
            pltpu.VMEM((HB * 3 * WP,), jnp.float32),
            pltpu.VMEM((HB * 3 * WP,), jnp.float32),
            pltpu.VMEM((L,), jnp.float32),
            pltpu.SemaphoreType.DMA,
        ],
    )
    return f(a, p_flat, n_flat, w)


def kernel(a, p, n, W):
    w3 = W.T.reshape(4, 8, 1000000)
    tail_flat = jax.lax.slice(W, (TAIL_START, 0), (1000000, D)).reshape(-1)
    wlin = _detile(w3, tail_flat)
    parts = _partial_losses(a, p.reshape(-1), n.reshape(-1),
                            wlin.reshape(1000000, D))
    return jnp.sum(parts) / jnp.float32(B)


# speedup vs baseline: 1.1283x; 1.1283x over previous
import functools

import jax
import jax.numpy as jnp
from jax import lax
from jax.experimental import pallas as pl
from jax.experimental.pallas import tpu as pltpu
from jax.experimental.pallas import tpu_sc as plsc

D = 32
B = 16384
L = 16

_info = plsc.get_sparse_core_info()
NC = _info.num_cores
NS = _info.num_subcores
NW = NC * NS
BPW = B // NW
GROUPS = BPW // L

MARGIN = 1.0
EPS = 1e-6


def _sqrt16(x):
    x = jnp.maximum(x, jnp.float32(1e-30))
    i = lax.bitcast_convert_type(x, jnp.int32)
    i = jnp.int32(0x1FBD1DF5) + lax.shift_right_arithmetic(i, 1)
    y = lax.bitcast_convert_type(i, jnp.float32)
    for _ in range(3):
        y = jnp.float32(0.5) * (y + x / y)
    return y


HB = BPW // 2
HG = GROUPS // 2
WP = 33


def _tec_body(a_hbm, p_hbm, n_hbm, w_hbm, out_hbm,
              idx_a, idx_p, idx_n, g_ea, g_ep, g_en, p_ea, p_ep, p_en,
              part_v, sem):
    wid = lax.axis_index("s") * NC + lax.axis_index("c")
    base = wid * BPW

    lanes = lax.iota(jnp.int32, L)
    loss_vec = jnp.zeros((L,), jnp.float32)

    for h in range(2):
        bh = base + h * HB
        pltpu.sync_copy(a_hbm.at[pl.ds(bh, HB)], idx_a)
        pltpu.sync_copy(p_hbm.at[pl.ds(bh * 3, HB * 3)], idx_p)
        pltpu.sync_copy(n_hbm.at[pl.ds(bh * 3, HB * 3)], idx_n)
        cp_a = pltpu.async_copy(w_hbm.at[idx_a], g_ea, sem)
        cp_p = pltpu.async_copy(w_hbm.at[idx_p], g_ep, sem)
        cp_n = pltpu.async_copy(w_hbm.at[idx_n], g_en, sem)
        cp_a.wait()
        cp_p.wait()
        cp_n.wait()

        for src_ref, dst_ref, nrows in ((g_ea, p_ea, HB),
                                        (g_ep, p_ep, HB * 3),
                                        (g_en, p_en, HB * 3)):
            @plsc.parallel_loop(0, nrows, step=4, unroll=4)
            def _(r, _src=src_ref, _dst=dst_ref):
                for u in range(4):
                    rr = r + u
                    _dst[pl.ds(rr * WP, L)] = _src[rr, pl.ds(0, L)]
                    _dst[pl.ds(rr * WP + L, L)] = _src[rr, pl.ds(L, L)]

        @plsc.parallel_loop(0, HG, carry=loss_vec)
        def loss_vec(g, acc_loss):
            rows_a = (g * L + lanes) * WP
            rows3 = (g * L + lanes) * 3 * WP
            xrows = (rows3, rows3 + WP, rows3 + 2 * WP,
                     rows3, rows3 + WP, rows3 + 2 * WP)
            xrefs = (p_ep, p_ep, p_ep, p_en, p_en, p_en)
            acc = [jnp.zeros((L,), jnp.float32) for _ in range(6)]
            for d in range(D):
                ea_d = plsc.load_gather(p_ea, [rows_a + d]) + jnp.float32(EPS)
                for j in range(6):
                    t = ea_d - plsc.load_gather(xrefs[j], [xrows[j] + d])
                    acc[j] = acc[j] + t * t
            dist = [_sqrt16(acc[j]) for j in range(6)]
            for k in range(5):
                acc_loss = acc_loss + jnp.maximum(
                    dist[k] - dist[k + 1] + jnp.float32(MARGIN),
                    jnp.float32(0.0))
            return acc_loss

    part_v[...] = loss_vec
    pltpu.sync_copy(part_v, out_hbm.at[wid])


CPW = 244
CHUNK = 4
STEPS = CPW // CHUNK
CN = CHUNK * 128
JG = CN // L
SP = 33
TAIL_START = CPW * NW * 128
TAIL_ROWS = 1000000 - TAIL_START
TPR = TAIL_ROWS // NW


def _detile_body(w3_hbm, tail_hbm, out_hbm, in0_v, in1_v, pad_v, pk0_v, pk1_v,
                 tail_v, isem0, isem1, osem0, osem1):
    ins = (in0_v, in1_v)
    pks = (pk0_v, pk1_v)
    isems = (isem0, isem1)
    osems = (osem0, osem1)
    wid = lax.axis_index("s") * NC + lax.axis_index("c")
    col0 = wid * CPW
    lanes_sp = lax.iota(jnp.int32, L) * SP

    def fire_in(g, b):
        c = (col0 + g * CHUNK) * 128
        pltpu.async_copy(w3_hbm.at[:, :, pl.ds(c, CN)], ins[b], isems[b])

    def wait_in(g, b):
        c = (col0 + g * CHUNK) * 128
        pltpu.make_async_copy(w3_hbm.at[:, :, pl.ds(c, CN)], ins[b],
                              isems[b]).wait()

    def fire_out(g, b):
        c = (col0 + g * CHUNK) * 128
        pltpu.async_copy(pk_v := pks[b], out_hbm.at[pl.ds(c * D, CN * D)],
                         osems[b])

    def wait_out(g, b):
        c = (col0 + g * CHUNK) * 128
        pltpu.make_async_copy(pks[b], out_hbm.at[pl.ds(c * D, CN * D)],
                              osems[b]).wait()

    def compute(b):
        @plsc.parallel_loop(0, JG, unroll=8)
        def _(j):
            jb = lanes_sp + j * (L * SP)
            for s in range(4):
                for d in range(8):
                    v = ins[b][s, d, pl.ds(j * L, L)]
                    plsc.store_scatter(pad_v, [jb + (s * 8 + d)], v)

        @plsc.parallel_loop(0, CN, step=4, unroll=4)
        def _(n):
            for u in range(4):
                nn = n + u
                pks[b][pl.ds(nn * D, L)] = pad_v[pl.ds(nn * SP, L)]
                pks[b][pl.ds(nn * D + L, L)] = pad_v[pl.ds(nn * SP + L, L)]

    fire_in(0, 0)
    fire_in(1, 1)

    def step2(i, carry):
        g = i * 2
        for b in range(2):
            ge = g + b
            wait_in(ge, b)

            @pl.when(ge >= 2)
            def _():
                wait_out(ge - 2, b)

            compute(b)
            fire_out(ge, b)

            @pl.when(ge + 2 < STEPS)
            def _():
                fire_in(ge + 2, b)
        return carry

    lax.fori_loop(0, (STEPS - 1) // 2, step2, 0)
    wait_in(STEPS - 1, 0)
    wait_out(STEPS - 3, 0)
    compute(0)
    fire_out(STEPS - 1, 0)
    wait_out(STEPS - 2, 1)
    wait_out(STEPS - 1, 0)
    pltpu.sync_copy(tail_hbm.at[pl.ds(wid * TPR * D, TPR * D)], tail_v)
    pltpu.sync_copy(tail_v,
                    out_hbm.at[pl.ds((TAIL_START + wid * TPR) * D, TPR * D)])


@functools.partial(jax.jit, static_argnums=())
def _detile(w3, tail_flat):
    mesh = plsc.VectorSubcoreMesh(core_axis_name="c", subcore_axis_name="s")
    f = pl.kernel(
        _detile_body,
        mesh=mesh,
        compiler_params=pltpu.CompilerParams(needs_layout_passes=False),
        out_type=jax.ShapeDtypeStruct((1000000 * D,), jnp.float32),
        scratch_types=[
            pltpu.VMEM((4, 8, CN), jnp.float32),
            pltpu.VMEM((4, 8, CN), jnp.float32),
            pltpu.VMEM((CN * SP,), jnp.float32),
            pltpu.VMEM((CN * D,), jnp.float32),
            pltpu.VMEM((CN * D,), jnp.float32),
            pltpu.VMEM((TPR * D,), jnp.float32),
            pltpu.SemaphoreType.DMA,
            pltpu.SemaphoreType.DMA,
            pltpu.SemaphoreType.DMA,
            pltpu.SemaphoreType.DMA,
        ],
    )
    return f(w3, tail_flat)


@functools.partial(jax.jit, static_argnums=())
def _partial_losses(a, p_flat, n_flat, w):
    mesh = plsc.VectorSubcoreMesh(core_axis_name="c", subcore_axis_name="s")
    f = pl.kernel(
        _tec_body,
        mesh=mesh,
        compiler_params=pltpu.CompilerParams(
            needs_layout_passes=False, use_tc_tiling_on_sc=False),
        out_type=jax.ShapeDtypeStruct((NW, L), jnp.float32),
        scratch_types=[
            pltpu.VMEM((HB,), jnp.int32),
            pltpu.VMEM((HB * 3,), jnp.int32),
            pltpu.VMEM((HB * 3,), jnp.int32),
            pltpu.VMEM((HB, D), jnp.float32),
            pltpu.VMEM((HB * 3, D), jnp.float32),
            pltpu.VMEM((HB * 3, D), jnp.float32),
            pltpu.VMEM((HB * WP,), jnp.float32),
            pltpu.VMEM((HB * 3 * WP,), jnp.float32),
            pltpu.VMEM((HB * 3 * WP,), jnp.float32),
            pltpu.VMEM((L,), jnp.float32),
            pltpu.SemaphoreType.DMA,
        ],
    )
    return f(a, p_flat, n_flat, w)


def kernel(a, p, n, W):
    w3 = W.T.reshape(4, 8, 1000000)
    tail_flat = jax.lax.slice(W, (TAIL_START, 0), (1000000, D)).reshape(-1)
    wlin = _detile(w3, tail_flat)
    parts = _partial_losses(a, p.reshape(-1), n.reshape(-1),
                            wlin.reshape(1000000, D))
    return jnp.sum(parts) / jnp.float32(B)
